# Initial kernel scaffold; baseline (speedup 1.0000x reference)
import jax
import jax.numpy as jnp
from jax.experimental import pallas as pl


def kernel(x_t, xA_pos, xB_pos, s, t, Z, edge_index, is_bond_A, is_bond_B, params):
    raise NotImplementedError("write your pallas kernel here")


# speedup vs baseline: 1.0008x; 1.0008x over previous
import jax
import jax.numpy as jnp
from jax.experimental import pallas as pl

N = 10000
D_CUTOFF = 5.0
N_FREQ = 8
N_RBF = 10
N_LAYERS = 2


def _mlp_apply(params, x):
    for i, (W, b) in enumerate(params):
        x = x @ W + b
        if i < len(params) - 1:
            x = jax.nn.gelu(x)
    return x


def _sinusoid(v, n_freq):
    k = 2.0 ** jnp.arange(n_freq, dtype=jnp.float32)
    ang = v[:, None] * k[None, :] * jnp.pi
    return jnp.concatenate([jnp.sin(ang), jnp.cos(ang)], axis=1)


def _rbf(d):
    centers = jnp.linspace(0.0, D_CUTOFF, N_RBF, dtype=jnp.float32)
    sigma = D_CUTOFF / N_RBF
    return jnp.exp(-((d - centers[None, :]) ** 2) / (2.0 * sigma * sigma))


def _center_kernel(x_ref, o_ref):
    x = x_ref[...]
    o_ref[...] = x - jnp.mean(x, axis=0, keepdims=True)


def kernel(x_t, xA_pos, xB_pos, s, t, Z, edge_index, is_bond_A, is_bond_B, params):
    zoh = jax.nn.one_hot(Z, 10, dtype=jnp.float32)
    atom_embedding = _mlp_apply(params["info"], zoh)
    hA = _mlp_apply(params["embA"], zoh)
    hB = _mlp_apply(params["embB"], zoh)
    s_embed = _sinusoid(s, N_FREQ)
    t_embed = _sinusoid(t, N_FREQ)
    h = jnp.concatenate([atom_embedding, hA, hB, s_embed, t_embed], axis=1)
    x = x_t
    src = edge_index[0]
    dst = edge_index[1]
    for l in range(N_LAYERS):
        dx = x[src] - x[dst]
        dist = jnp.sqrt(jnp.sum(dx * dx, axis=1, keepdims=True) + 1e-12)
        rbf = _rbf(dist)
        dxA = xA_pos[src] - xA_pos[dst]
        dist_xA = jnp.sqrt(jnp.sum(dxA * dxA, axis=1, keepdims=True) + 1e-12)
        rbf_A = _rbf(dist_xA)
        dxB = xB_pos[src] - xB_pos[dst]
        dist_xB = jnp.sqrt(jnp.sum(dxB * dxB, axis=1, keepdims=True) + 1e-12)
        rbf_B = _rbf(dist_xB)
        edge_features = jnp.concatenate([is_bond_A[:, None], is_bond_B[:, None], dist, dist ** 2,
                                         dist_xA, dist_xB, dist_xA - dist_xB, rbf, rbf_A, rbf_B], axis=1)
        message_inputs = jnp.concatenate([h[src], h[dst], edge_features], axis=1)
        messages = _mlp_apply(params["message"][l], message_inputs)
        node_messages = jax.ops.segment_sum(messages, dst, num_segments=N)
        h = h + _mlp_apply(params["state"][l], jnp.concatenate([h, node_messages], axis=1))
        edge_inputs = jnp.concatenate([h[src], h[dst], edge_features], axis=1)
        alpha = _mlp_apply(params["alpha"][l], edge_inputs)
        beta = _mlp_apply(params["beta"][l], h)
        gamma = _mlp_apply(params["gamma"][l], h)
        neighbor_update = jax.ops.segment_sum(alpha * dx, dst, num_segments=N)
        x = x + neighbor_update + beta * (1.0 - s[:, None]) * (xA_pos - x) + gamma * s[:, None] * (xB_pos - x)
    return pl.pallas_call(
        _center_kernel,
        out_shape=jax.ShapeDtypeStruct(x.shape, x.dtype),
    )(x)


# speedup vs baseline: 2.6258x; 2.6238x over previous
import functools

import jax
import jax.numpy as jnp
from jax import lax
from jax.experimental import pallas as pl
from jax.experimental.pallas import tpu as pltpu
from jax.experimental.pallas import tpu_sc as plsc

F32 = jnp.float32
N = 10000
E = 160000
STATE = 224
NFREQ = 8
NRBF = 10
DCUT = 5.0

NW = 32
NC = 2
NS = 16
LCH = 128
E_PAD = 163840
CPW = E_PAD // (NW * LCH)
N_ACC = 10112
BE = 2048
BN = 1000


def _gelu(x):
    return jax.nn.gelu(x)


def _make_sc_gather(width):
    mesh = plsc.VectorSubcoreMesh(core_axis_name="c", subcore_axis_name="s",
                                  num_cores=NC, num_subcores=NS)

    @functools.partial(
        pl.kernel,
        out_type=(jax.ShapeDtypeStruct((E_PAD, width), F32),
                  jax.ShapeDtypeStruct((E_PAD, width), F32)),
        mesh=mesh,
        scratch_types=[
            pltpu.VMEM((CPW, LCH), jnp.int32),
            pltpu.VMEM((CPW, LCH), jnp.int32),
            pltpu.VMEM((LCH, width), F32),
            pltpu.VMEM((LCH, width), F32),
            pltpu.SemaphoreType.DMA,
            pltpu.SemaphoreType.DMA,
        ],
        name=f"sc_gather{width}",
    )
    def gather(tab_s, tab_d, srcw, dstw, out_s, out_d,
               idxs, idxd, bufs, bufd, sem1, sem2):
        wid = lax.axis_index("s") * NC + lax.axis_index("c")
        pltpu.sync_copy(srcw.at[wid], idxs)
        pltpu.sync_copy(dstw.at[wid], idxd)

        def body(j, carry):
            base = (wid * CPW + j) * LCH
            c1 = pltpu.async_copy(tab_s.at[idxs.at[j]], bufs, sem1)
            c2 = pltpu.async_copy(tab_d.at[idxd.at[j]], bufd, sem2)
            c1.wait()
            c2.wait()
            pltpu.sync_copy(bufs, out_s.at[pl.ds(base, LCH)])
            pltpu.sync_copy(bufd, out_d.at[pl.ds(base, LCH)])
            return carry

        lax.fori_loop(0, CPW, body, 0)

    return gather


def _make_sc_scatter(width):
    del width
    mesh = plsc.VectorSubcoreMesh(core_axis_name="c", subcore_axis_name="s",
                                  num_cores=NC, num_subcores=NS)
    rows = N_ACC // NS

    @functools.partial(
        pl.kernel,
        out_type=jax.ShapeDtypeStruct((2, N_ACC, 128), F32),
        mesh=mesh,
        scratch_types=[
            pltpu.VMEM((LCH,), jnp.int32),
            pltpu.VMEM((LCH, 128), F32),
            pltpu.VMEM_SHARED((N_ACC, 128), F32),
        ],
        name="sc_scatter128",
    )
    def scatter(vals, dstw, zeros_hbm, out, idxc, buf, acc):
        cid = lax.axis_index("c")
        sid = lax.axis_index("s")
        wid = sid * NC + cid
        pltpu.sync_copy(zeros_hbm.at[pl.ds(sid * rows, rows)],
                        acc.at[pl.ds(sid * rows, rows)])
        plsc.subcore_barrier()

        def body(j, carry):
            base = (wid * CPW + j) * LCH
            pltpu.sync_copy(dstw.at[wid, j], idxc)
            pltpu.sync_copy(vals.at[pl.ds(base, LCH)], buf)
            pltpu.sync_copy(buf, acc.at[idxc], add=True)
            return carry

        lax.fori_loop(0, CPW, body, 0)
        plsc.subcore_barrier()
        pltpu.sync_copy(acc.at[pl.ds(sid * rows, rows)],
                        out.at[cid, pl.ds(sid * rows, rows)])

    return scatter


_get_gather = functools.lru_cache(None)(_make_sc_gather)
_get_scatter = functools.lru_cache(None)(_make_sc_scatter)


def _full(a):
    return pl.BlockSpec(a.shape, lambda i: (0,) * a.ndim)


def _rbf_feats(d):
    c = lax.broadcasted_iota(jnp.int32, (1, NRBF), 1).astype(F32) * (DCUT / (NRBF - 1))
    inv2s2 = 1.0 / (2.0 * (DCUT / NRBF) ** 2)
    return jnp.exp(-((d - c) ** 2) * inv2s2)


def _edge_feats(gs, gd, ib2):
    xs = gs[:, 64:73]
    xd = gd[:, 64:73]
    dxyz = xs - xd
    d0 = dxyz[:, 0:3]
    d2 = jnp.sum(d0 * d0, axis=1, keepdims=True)
    dist = jnp.sqrt(d2 + 1e-12)
    dA3 = dxyz[:, 3:6]
    dA = jnp.sqrt(jnp.sum(dA3 * dA3, axis=1, keepdims=True) + 1e-12)
    dB3 = dxyz[:, 6:9]
    dB = jnp.sqrt(jnp.sum(dB3 * dB3, axis=1, keepdims=True) + 1e-12)
    ef = jnp.concatenate([ib2, dist, dist * dist, dA, dB, dA - dB,
                          _rbf_feats(dist), _rbf_feats(dA), _rbf_feats(dB)],
                         axis=1)
    return ef, d0


def _edge_msg_body(gs_ref, gd_ref, ib_ref, w1e, b1, w2, b2, w3, b3, out_ref):
    gs = gs_ref[...]
    gd = gd_ref[...]
    ef, _ = _edge_feats(gs, gd, ib_ref[...])
    g = gs[:, :64] + gd[:, :64]
    u = _gelu(g + jnp.dot(ef, w1e[...]) + b1[...])
    u = _gelu(jnp.dot(u, w2[...]) + b2[...])
    m = jnp.dot(u, w3[...]) + b3[...]
    out_ref[...] = jnp.concatenate([m, jnp.zeros((m.shape[0], 64), F32)], axis=1)


def _edge_alpha_body(ga_ref, gb_ref, gs_ref, gd_ref, ib_ref,
                     w1e, b1, w2, b2, w3, b3, out_ref):
    gs = gs_ref[...]
    gd = gd_ref[...]
    ef, d0 = _edge_feats(gs, gd, ib_ref[...])
    g = ga_ref[:, :64] + gb_ref[:, 64:128]
    u = _gelu(g + jnp.dot(ef, w1e[...]) + b1[...])
    u = _gelu(jnp.dot(u, w2[...]) + b2[...])
    alpha = jnp.dot(u, w3[...]) + b3[...]
    av = alpha * d0
    out_ref[...] = jnp.concatenate(
        [av, jnp.zeros((av.shape[0], 125), F32)], axis=1)


def _prologue_body(zf_ref, s_ref, t_ref, x_ref, xa_ref, xb_ref,
                   wi1, bi1, wi2, bi2, wa1, ba1, wa2, ba2,
                   wb1, bb1, wb2, bb2, wms, wmd,
                   h_ref, ts_ref, td_ref):
    tab = jnp.concatenate([
        jnp.dot(_gelu(wi1[...] + bi1[...]), wi2[...]) + bi2[...],
        jnp.dot(_gelu(wa1[...] + ba1[...]), wa2[...]) + ba2[...],
        jnp.dot(_gelu(wb1[...] + bb1[...]), wb2[...]) + bb2[...],
    ], axis=1)
    z = zf_ref[...]
    oh = (z == lax.broadcasted_iota(jnp.int32, (z.shape[0], 10), 1).astype(F32)).astype(F32)
    emb = jnp.dot(oh, tab)
    k = jnp.exp2(lax.broadcasted_iota(jnp.int32, (1, NFREQ), 1).astype(F32)) * jnp.pi
    angs = s_ref[...] * k
    angt = t_ref[...] * k
    h = jnp.concatenate([emb, jnp.sin(angs), jnp.cos(angs),
                         jnp.sin(angt), jnp.cos(angt)], axis=1)
    h_ref[...] = h
    p = jnp.concatenate([x_ref[...], xa_ref[...], xb_ref[...],
                         jnp.zeros((h.shape[0], 55), F32)], axis=1)
    ts_ref[...] = jnp.concatenate([jnp.dot(h, wms[...]), p], axis=1)
    td_ref[...] = jnp.concatenate([jnp.dot(h, wmd[...]), p], axis=1)


def _node_update_body0(h_ref, n0_ref, n1_ref,
                       ws1a, ws1b, bs1, ws2, bs2, ws3, bs3,
                       wa1s, wa1d,
                       wbe1, bbe1, wbe2, bbe2, wbe3, bbe3,
                       wg1, bg1, wg2, bg2, wg3, bg3,
                       wmsn, wmdn,
                       h_out, ta_out, bg_out, msn_out, mdn_out):
    h = h_ref[...]
    nm = n0_ref[:, :64] + n1_ref[:, :64]
    v = _gelu(jnp.dot(h, ws1a[...]) + jnp.dot(nm, ws1b[...]) + bs1[...])
    v = _gelu(jnp.dot(v, ws2[...]) + bs2[...])
    hn = h + jnp.dot(v, ws3[...]) + bs3[...]
    h_out[...] = hn
    ta_out[...] = jnp.concatenate(
        [jnp.dot(hn, wa1s[...]), jnp.dot(hn, wa1d[...])], axis=1)
    ub = _gelu(jnp.dot(hn, wbe1[...]) + bbe1[...])
    ub = _gelu(jnp.dot(ub, wbe2[...]) + bbe2[...])
    beta = jnp.dot(ub, wbe3[...]) + bbe3[...]
    ug = _gelu(jnp.dot(hn, wg1[...]) + bg1[...])
    ug = _gelu(jnp.dot(ug, wg2[...]) + bg2[...])
    gamma = jnp.dot(ug, wg3[...]) + bg3[...]
    bg_out[...] = jnp.concatenate([beta, gamma], axis=1)
    msn_out[...] = jnp.dot(hn, wmsn[...])
    mdn_out[...] = jnp.dot(hn, wmdn[...])


def _node_update_body1(h_ref, n0_ref, n1_ref,
                       ws1a, ws1b, bs1, ws2, bs2, ws3, bs3,
                       wa1s, wa1d,
                       wbe1, bbe1, wbe2, bbe2, wbe3, bbe3,
                       wg1, bg1, wg2, bg2, wg3, bg3,
                       h_out, ta_out, bg_out):
    h = h_ref[...]
    nm = n0_ref[:, :64] + n1_ref[:, :64]
    v = _gelu(jnp.dot(h, ws1a[...]) + jnp.dot(nm, ws1b[...]) + bs1[...])
    v = _gelu(jnp.dot(v, ws2[...]) + bs2[...])
    hn = h + jnp.dot(v, ws3[...]) + bs3[...]
    h_out[...] = hn
    ta_out[...] = jnp.concatenate(
        [jnp.dot(hn, wa1s[...]), jnp.dot(hn, wa1d[...])], axis=1)
    ub = _gelu(jnp.dot(hn, wbe1[...]) + bbe1[...])
    ub = _gelu(jnp.dot(ub, wbe2[...]) + bbe2[...])
    beta = jnp.dot(ub, wbe3[...]) + bbe3[...]
    ug = _gelu(jnp.dot(hn, wg1[...]) + bg1[...])
    ug = _gelu(jnp.dot(ug, wg2[...]) + bg2[...])
    gamma = jnp.dot(ug, wg3[...]) + bg3[...]
    bg_out[...] = jnp.concatenate([beta, gamma], axis=1)


def _x_update(x, xa, xb, s2, bg, nu):
    beta = bg[:, 0:1]
    gamma = bg[:, 1:2]
    return x + nu + beta * (1.0 - s2) * (xa - x) + gamma * s2 * (xb - x)


def _transition_body(x_ref, xa_ref, xb_ref, s_ref, bg_ref, n0_ref, n1_ref,
                     msn_ref, mdn_ref, ts_ref, td_ref, x_out):
    nu = n0_ref[:, 0:3] + n1_ref[:, 0:3]
    xn = _x_update(x_ref[...], xa_ref[...], xb_ref[...], s_ref[...],
                   bg_ref[...], nu)
    x_out[...] = xn
    p = jnp.concatenate([xn, xa_ref[...], xb_ref[...],
                         jnp.zeros((xn.shape[0], 55), F32)], axis=1)
    ts_ref[...] = jnp.concatenate([msn_ref[...], p], axis=1)
    td_ref[...] = jnp.concatenate([mdn_ref[...], p], axis=1)


def _final_body(x_ref, xa_ref, xb_ref, s_ref, bg_ref, n0_ref, n1_ref, out_ref):
    nu = n0_ref[:, 0:3] + n1_ref[:, 0:3]
    xn = _x_update(x_ref[...], xa_ref[...], xb_ref[...], s_ref[...],
                   bg_ref[...], nu)
    out_ref[...] = xn - jnp.mean(xn, axis=0, keepdims=True)


def _edge_spec(width):
    return pl.BlockSpec((BE, width), lambda i: (i, 0))


def _node_spec(width):
    return pl.BlockSpec((BN, width), lambda i: (i, 0))


def kernel(x_t, xA_pos, xB_pos, s, t, Z, edge_index, is_bond_A, is_bond_B, params):
    src = edge_index[0].astype(jnp.int32)
    dst = edge_index[1].astype(jnp.int32)
    pad = E_PAD - E
    srcp = jnp.concatenate([src, jnp.zeros((pad,), jnp.int32)]).reshape(NW, CPW, LCH)
    dstp = jnp.concatenate([dst, jnp.zeros((pad,), jnp.int32)]).reshape(NW, CPW, LCH)
    dsts = jnp.concatenate([dst, jnp.full((pad,), N, jnp.int32)]).reshape(NW, CPW, LCH)
    ib2 = jnp.concatenate(
        [jnp.stack([is_bond_A, is_bond_B], axis=1), jnp.zeros((pad, 2), F32)], axis=0)
    zeros128 = jnp.zeros((N_ACC, 128), F32)
    s2 = s[:, None]
    t2 = t[:, None]
    zf = Z.astype(F32)[:, None]

    def w2d(b):
        return b.reshape(1, -1)

    P = params
    NG = N // BN
    EG = E_PAD // BE

    wm0, bm0 = P["message"][0][0]
    pro_in = [zf, s2, t2, x_t, xA_pos, xB_pos,
              P["info"][0][0], w2d(P["info"][0][1]), P["info"][1][0], w2d(P["info"][1][1]),
              P["embA"][0][0], w2d(P["embA"][0][1]), P["embA"][1][0], w2d(P["embA"][1][1]),
              P["embB"][0][0], w2d(P["embB"][0][1]), P["embB"][1][0], w2d(P["embB"][1][1]),
              wm0[:STATE], wm0[STATE:2 * STATE]]
    h, tabs, tabd = pl.pallas_call(
        _prologue_body,
        grid=(NG,),
        in_specs=[_node_spec(1), _node_spec(1), _node_spec(1),
                  _node_spec(3), _node_spec(3), _node_spec(3)] +
                 [_full(a) for a in pro_in[6:]],
        out_specs=[_node_spec(STATE), _node_spec(128), _node_spec(128)],
        out_shape=[jax.ShapeDtypeStruct((N, STATE), F32),
                   jax.ShapeDtypeStruct((N, 128), F32),
                   jax.ShapeDtypeStruct((N, 128), F32)],
        name="prologue",
    )(*pro_in)

    x = x_t
    out = None
    for l in range(2):
        wm1, bm1 = P["message"][l][0]
        wm2, bm2 = P["message"][l][1]
        wm3, bm3 = P["message"][l][2]
        ws1, bs1 = P["state"][l][0]
        ws2, bs2 = P["state"][l][1]
        ws3, bs3 = P["state"][l][2]
        wa1, ba1 = P["alpha"][l][0]
        wa2, ba2 = P["alpha"][l][1]
        wa3, ba3 = P["alpha"][l][2]

        gs, gd = _get_gather(128)(tabs, tabd, srcp, dstp)

        msg_w = [wm1[2 * STATE:], w2d(bm1), wm2, w2d(bm2), wm3, w2d(bm3)]
        msg = pl.pallas_call(
            _edge_msg_body,
            grid=(EG,),
            in_specs=[_edge_spec(128), _edge_spec(128), _edge_spec(2)] +
                     [_full(a) for a in msg_w],
            out_specs=_edge_spec(128),
            out_shape=jax.ShapeDtypeStruct((E_PAD, 128), F32),
            name="edge_msg",
        )(gs, gd, ib2, *msg_w)

        nmp = _get_scatter(128)(msg, dsts, zeros128)
        nm0, nm1 = nmp[0], nmp[1]

        node_w = [ws1[:STATE], ws1[STATE:], w2d(bs1), ws2, w2d(bs2), ws3, w2d(bs3),
                  wa1[:STATE], wa1[STATE:2 * STATE]]
        for nm_ in ("beta", "gamma"):
            for li in range(3):
                node_w.append(P[nm_][l][li][0])
                node_w.append(w2d(P[nm_][l][li][1]))
        if l == 0:
            wmn = P["message"][1][0][0]
            node_w += [wmn[:STATE], wmn[STATE:2 * STATE]]
            h, taba, bgv, msn, mdn = pl.pallas_call(
                _node_update_body0,
                grid=(NG,),
                in_specs=[_node_spec(STATE),
                          pl.BlockSpec((BN, 128), lambda i: (i, 0)),
                          pl.BlockSpec((BN, 128), lambda i: (i, 0))] +
                         [_full(a) for a in node_w],
                out_specs=[_node_spec(STATE), _node_spec(128),
                           _node_spec(2), _node_spec(64), _node_spec(64)],
                out_shape=[jax.ShapeDtypeStruct((N, STATE), F32),
                           jax.ShapeDtypeStruct((N, 128), F32),
                           jax.ShapeDtypeStruct((N, 2), F32),
                           jax.ShapeDtypeStruct((N, 64), F32),
                           jax.ShapeDtypeStruct((N, 64), F32)],
                name="node_update0",
            )(h, nm0, nm1, *node_w)
        else:
            h, taba, bgv = pl.pallas_call(
                _node_update_body1,
                grid=(NG,),
                in_specs=[_node_spec(STATE),
                          pl.BlockSpec((BN, 128), lambda i: (i, 0)),
                          pl.BlockSpec((BN, 128), lambda i: (i, 0))] +
                         [_full(a) for a in node_w],
                out_specs=[_node_spec(STATE), _node_spec(128),
                           _node_spec(2)],
                out_shape=[jax.ShapeDtypeStruct((N, STATE), F32),
                           jax.ShapeDtypeStruct((N, 128), F32),
                           jax.ShapeDtypeStruct((N, 2), F32)],
                name="node_update1",
            )(h, nm0, nm1, *node_w)

        ga, gb = _get_gather(128)(taba, taba, srcp, dstp)

        al_w = [wa1[2 * STATE:], w2d(ba1), wa2, w2d(ba2), wa3, w2d(ba3)]
        av = pl.pallas_call(
            _edge_alpha_body,
            grid=(EG,),
            in_specs=[_edge_spec(128), _edge_spec(128), _edge_spec(128),
                      _edge_spec(128), _edge_spec(2)] +
                     [_full(a) for a in al_w],
            out_specs=_edge_spec(128),
            out_shape=jax.ShapeDtypeStruct((E_PAD, 128), F32),
            name="edge_alpha",
        )(ga, gb, gs, gd, ib2, *al_w)

        nup = _get_scatter(128)(av, dsts, zeros128)
        nu0, nu1 = nup[0], nup[1]

        if l == 0:
            tabs, tabd, x = pl.pallas_call(
                _transition_body,
                grid=(NG,),
                in_specs=[_node_spec(3), _node_spec(3), _node_spec(3),
                          _node_spec(1), _node_spec(2),
                          pl.BlockSpec((BN, 128), lambda i: (i, 0)),
                          pl.BlockSpec((BN, 128), lambda i: (i, 0)),
                          _node_spec(64), _node_spec(64)],
                out_specs=[_node_spec(128), _node_spec(128), _node_spec(3)],
                out_shape=[jax.ShapeDtypeStruct((N, 128), F32),
                           jax.ShapeDtypeStruct((N, 128), F32),
                           jax.ShapeDtypeStruct((N, 3), F32)],
                name="transition",
            )(x, xA_pos, xB_pos, s2, bgv, nu0, nu1, msn, mdn)
        else:
            out = pl.pallas_call(
                _final_body,
                grid=(1,),
                in_specs=[pl.BlockSpec((N, 3), lambda i: (0, 0)),
                          pl.BlockSpec((N, 3), lambda i: (0, 0)),
                          pl.BlockSpec((N, 3), lambda i: (0, 0)),
                          pl.BlockSpec((N, 1), lambda i: (0, 0)),
                          pl.BlockSpec((N, 2), lambda i: (0, 0)),
                          pl.BlockSpec((N, 128), lambda i: (0, 0)),
                          pl.BlockSpec((N, 128), lambda i: (0, 0))],
                out_specs=pl.BlockSpec((N, 3), lambda i: (0, 0)),
                out_shape=jax.ShapeDtypeStruct((N, 3), F32),
                name="final",
            )(x, xA_pos, xB_pos, s2, bgv, nu0, nu1)
    return out


# speedup vs baseline: 2.7869x; 1.0614x over previous
import functools

import jax
import jax.numpy as jnp
from jax import lax
from jax.experimental import pallas as pl
from jax.experimental.pallas import tpu as pltpu
from jax.experimental.pallas import tpu_sc as plsc

F32 = jnp.float32
N = 10000
E = 160000
STATE = 224
NFREQ = 8
NRBF = 10
DCUT = 5.0

NW = 32
NC = 2
NS = 16
LCH = 128
E_PAD = 163840
CPW = E_PAD // (NW * LCH)
N_ACC = 10112
BE = 2048
BN = 1000


def _gelu(x):
    return jax.nn.gelu(x)


def _make_sc_gather(width):
    mesh = plsc.VectorSubcoreMesh(core_axis_name="c", subcore_axis_name="s",
                                  num_cores=NC, num_subcores=NS)

    @functools.partial(
        pl.kernel,
        out_type=(jax.ShapeDtypeStruct((E_PAD, width), F32),
                  jax.ShapeDtypeStruct((E_PAD, width), F32)),
        mesh=mesh,
        scratch_types=[
            pltpu.VMEM((CPW, LCH), jnp.int32),
            pltpu.VMEM((CPW, LCH), jnp.int32),
            pltpu.VMEM((2, LCH, width), F32),
            pltpu.VMEM((2, LCH, width), F32),
        ] + [pltpu.SemaphoreType.DMA] * 8,
        name=f"sc_gather{width}",
    )
    def gather(tab_s, tab_d, srcw, dstw, out_s, out_d,
               idxs, idxd, bufs, bufd,
               gs0, gs1, gd0, gd1, os0, os1, od0, od1):
        wid = lax.axis_index("s") * NC + lax.axis_index("c")
        gsem = (gs0, gs1)
        dsem = (gd0, gd1)
        osem = (os0, os1)
        psem = (od0, od1)
        pltpu.sync_copy(srcw.at[wid], idxs)
        pltpu.sync_copy(dstw.at[wid], idxd)
        for b in range(2):
            pltpu.async_copy(tab_s.at[idxs.at[b]], bufs.at[b], gsem[b])
            pltpu.async_copy(tab_d.at[idxd.at[b]], bufd.at[b], dsem[b])

        def body(p, carry):
            for b in range(2):
                jc = 2 * p + b
                base = (wid * CPW + jc) * LCH
                pltpu.make_async_copy(tab_s.at[idxs.at[jc]],
                                      bufs.at[b], gsem[b]).wait()
                pltpu.make_async_copy(tab_d.at[idxd.at[jc]],
                                      bufd.at[b], dsem[b]).wait()
                pltpu.async_copy(bufs.at[b], out_s.at[pl.ds(base, LCH)],
                                 osem[b])
                pltpu.async_copy(bufd.at[b], out_d.at[pl.ds(base, LCH)],
                                 psem[b])

                @pl.when(p < CPW // 2 - 1)
                def _():
                    pltpu.make_async_copy(bufs.at[b],
                                          out_s.at[pl.ds(base, LCH)],
                                          osem[b]).wait()
                    pltpu.make_async_copy(bufd.at[b],
                                          out_d.at[pl.ds(base, LCH)],
                                          psem[b]).wait()
                    pltpu.async_copy(tab_s.at[idxs.at[jc + 2]],
                                     bufs.at[b], gsem[b])
                    pltpu.async_copy(tab_d.at[idxd.at[jc + 2]],
                                     bufd.at[b], dsem[b])
            return carry

        lax.fori_loop(0, CPW // 2, body, 0)
        for b in range(2):
            base = (wid * CPW + CPW - 2 + b) * LCH
            pltpu.make_async_copy(bufs.at[b], out_s.at[pl.ds(base, LCH)],
                                  osem[b]).wait()
            pltpu.make_async_copy(bufd.at[b], out_d.at[pl.ds(base, LCH)],
                                  psem[b]).wait()

    return gather


def _make_sc_scatter(width):
    del width
    mesh = plsc.VectorSubcoreMesh(core_axis_name="c", subcore_axis_name="s",
                                  num_cores=NC, num_subcores=NS)
    rows = N_ACC // NS

    @functools.partial(
        pl.kernel,
        out_type=jax.ShapeDtypeStruct((2, N_ACC, 128), F32),
        mesh=mesh,
        scratch_types=[
            pltpu.VMEM((LCH,), jnp.int32),
            pltpu.VMEM((LCH, 128), F32),
            pltpu.VMEM_SHARED((N_ACC, 128), F32),
        ],
        name="sc_scatter128",
    )
    def scatter(vals, dstw, zeros_hbm, out, idxc, buf, acc):
        cid = lax.axis_index("c")
        sid = lax.axis_index("s")
        wid = sid * NC + cid
        pltpu.sync_copy(zeros_hbm.at[pl.ds(sid * rows, rows)],
                        acc.at[pl.ds(sid * rows, rows)])
        plsc.subcore_barrier()

        def body(j, carry):
            base = (wid * CPW + j) * LCH
            pltpu.sync_copy(dstw.at[wid, j], idxc)
            pltpu.sync_copy(vals.at[pl.ds(base, LCH)], buf)
            pltpu.sync_copy(buf, acc.at[idxc], add=True)
            return carry

        lax.fori_loop(0, CPW, body, 0)
        plsc.subcore_barrier()
        pltpu.sync_copy(acc.at[pl.ds(sid * rows, rows)],
                        out.at[cid, pl.ds(sid * rows, rows)])

    return scatter


_get_gather = functools.lru_cache(None)(_make_sc_gather)
_get_scatter = functools.lru_cache(None)(_make_sc_scatter)


def _full(a):
    return pl.BlockSpec(a.shape, lambda i: (0,) * a.ndim)


def _rbf_feats(d):
    c = lax.broadcasted_iota(jnp.int32, (1, NRBF), 1).astype(F32) * (DCUT / (NRBF - 1))
    inv2s2 = 1.0 / (2.0 * (DCUT / NRBF) ** 2)
    return jnp.exp(-((d - c) ** 2) * inv2s2)


def _edge_feats(gs, gd, ib2):
    xs = gs[:, 64:73]
    xd = gd[:, 64:73]
    dxyz = xs - xd
    d0 = dxyz[:, 0:3]
    d2 = jnp.sum(d0 * d0, axis=1, keepdims=True)
    dist = jnp.sqrt(d2 + 1e-12)
    dA3 = dxyz[:, 3:6]
    dA = jnp.sqrt(jnp.sum(dA3 * dA3, axis=1, keepdims=True) + 1e-12)
    dB3 = dxyz[:, 6:9]
    dB = jnp.sqrt(jnp.sum(dB3 * dB3, axis=1, keepdims=True) + 1e-12)
    ef = jnp.concatenate([ib2, dist, dist * dist, dA, dB, dA - dB,
                          _rbf_feats(dist), _rbf_feats(dA), _rbf_feats(dB)],
                         axis=1)
    return ef, d0


def _edge_msg_body(gs_ref, gd_ref, ib_ref, w1e, b1, w2, b2, w3, b3, out_ref):
    gs = gs_ref[...]
    gd = gd_ref[...]
    ef, _ = _edge_feats(gs, gd, ib_ref[...])
    g = gs[:, :64] + gd[:, :64]
    u = _gelu(g + jnp.dot(ef, w1e[...]) + b1[...])
    u = _gelu(jnp.dot(u, w2[...]) + b2[...])
    m = jnp.dot(u, w3[...]) + b3[...]
    out_ref[...] = jnp.concatenate([m, jnp.zeros((m.shape[0], 64), F32)], axis=1)


def _edge_alpha_body(ga_ref, gb_ref, gs_ref, gd_ref, ib_ref,
                     w1e, b1, w2, b2, w3, b3, out_ref):
    gs = gs_ref[...]
    gd = gd_ref[...]
    ef, d0 = _edge_feats(gs, gd, ib_ref[...])
    g = ga_ref[:, :64] + gb_ref[:, 64:128]
    u = _gelu(g + jnp.dot(ef, w1e[...]) + b1[...])
    u = _gelu(jnp.dot(u, w2[...]) + b2[...])
    alpha = jnp.dot(u, w3[...]) + b3[...]
    av = alpha * d0
    out_ref[...] = jnp.concatenate(
        [av, jnp.zeros((av.shape[0], 125), F32)], axis=1)


def _prologue_body(zf_ref, s_ref, t_ref, x_ref, xa_ref, xb_ref,
                   wi1, bi1, wi2, bi2, wa1, ba1, wa2, ba2,
                   wb1, bb1, wb2, bb2, wms, wmd,
                   h_ref, ts_ref, td_ref):
    tab = jnp.concatenate([
        jnp.dot(_gelu(wi1[...] + bi1[...]), wi2[...]) + bi2[...],
        jnp.dot(_gelu(wa1[...] + ba1[...]), wa2[...]) + ba2[...],
        jnp.dot(_gelu(wb1[...] + bb1[...]), wb2[...]) + bb2[...],
    ], axis=1)
    z = zf_ref[...]
    oh = (z == lax.broadcasted_iota(jnp.int32, (z.shape[0], 10), 1).astype(F32)).astype(F32)
    emb = jnp.dot(oh, tab)
    k = jnp.exp2(lax.broadcasted_iota(jnp.int32, (1, NFREQ), 1).astype(F32)) * jnp.pi
    angs = s_ref[...] * k
    angt = t_ref[...] * k
    h = jnp.concatenate([emb, jnp.sin(angs), jnp.cos(angs),
                         jnp.sin(angt), jnp.cos(angt)], axis=1)
    h_ref[...] = h
    p = jnp.concatenate([x_ref[...], xa_ref[...], xb_ref[...],
                         jnp.zeros((h.shape[0], 55), F32)], axis=1)
    ts_ref[...] = jnp.concatenate([jnp.dot(h, wms[...]), p], axis=1)
    td_ref[...] = jnp.concatenate([jnp.dot(h, wmd[...]), p], axis=1)


def _node_update_body0(h_ref, n0_ref, n1_ref,
                       ws1a, ws1b, bs1, ws2, bs2, ws3, bs3,
                       wa1s, wa1d,
                       wbe1, bbe1, wbe2, bbe2, wbe3, bbe3,
                       wg1, bg1, wg2, bg2, wg3, bg3,
                       wmsn, wmdn,
                       h_out, ta_out, bg_out, msn_out, mdn_out):
    h = h_ref[...]
    nm = n0_ref[:, :64] + n1_ref[:, :64]
    v = _gelu(jnp.dot(h, ws1a[...]) + jnp.dot(nm, ws1b[...]) + bs1[...])
    v = _gelu(jnp.dot(v, ws2[...]) + bs2[...])
    hn = h + jnp.dot(v, ws3[...]) + bs3[...]
    h_out[...] = hn
    ta_out[...] = jnp.concatenate(
        [jnp.dot(hn, wa1s[...]), jnp.dot(hn, wa1d[...])], axis=1)
    ub = _gelu(jnp.dot(hn, wbe1[...]) + bbe1[...])
    ub = _gelu(jnp.dot(ub, wbe2[...]) + bbe2[...])
    beta = jnp.dot(ub, wbe3[...]) + bbe3[...]
    ug = _gelu(jnp.dot(hn, wg1[...]) + bg1[...])
    ug = _gelu(jnp.dot(ug, wg2[...]) + bg2[...])
    gamma = jnp.dot(ug, wg3[...]) + bg3[...]
    bg_out[...] = jnp.concatenate([beta, gamma], axis=1)
    msn_out[...] = jnp.dot(hn, wmsn[...])
    mdn_out[...] = jnp.dot(hn, wmdn[...])


def _node_update_body1(h_ref, n0_ref, n1_ref,
                       ws1a, ws1b, bs1, ws2, bs2, ws3, bs3,
                       wa1s, wa1d,
                       wbe1, bbe1, wbe2, bbe2, wbe3, bbe3,
                       wg1, bg1, wg2, bg2, wg3, bg3,
                       h_out, ta_out, bg_out):
    h = h_ref[...]
    nm = n0_ref[:, :64] + n1_ref[:, :64]
    v = _gelu(jnp.dot(h, ws1a[...]) + jnp.dot(nm, ws1b[...]) + bs1[...])
    v = _gelu(jnp.dot(v, ws2[...]) + bs2[...])
    hn = h + jnp.dot(v, ws3[...]) + bs3[...]
    h_out[...] = hn
    ta_out[...] = jnp.concatenate(
        [jnp.dot(hn, wa1s[...]), jnp.dot(hn, wa1d[...])], axis=1)
    ub = _gelu(jnp.dot(hn, wbe1[...]) + bbe1[...])
    ub = _gelu(jnp.dot(ub, wbe2[...]) + bbe2[...])
    beta = jnp.dot(ub, wbe3[...]) + bbe3[...]
    ug = _gelu(jnp.dot(hn, wg1[...]) + bg1[...])
    ug = _gelu(jnp.dot(ug, wg2[...]) + bg2[...])
    gamma = jnp.dot(ug, wg3[...]) + bg3[...]
    bg_out[...] = jnp.concatenate([beta, gamma], axis=1)


def _x_update(x, xa, xb, s2, bg, nu):
    beta = bg[:, 0:1]
    gamma = bg[:, 1:2]
    return x + nu + beta * (1.0 - s2) * (xa - x) + gamma * s2 * (xb - x)


def _transition_body(x_ref, xa_ref, xb_ref, s_ref, bg_ref, n0_ref, n1_ref,
                     msn_ref, mdn_ref, ts_ref, td_ref, x_out):
    nu = n0_ref[:, 0:3] + n1_ref[:, 0:3]
    xn = _x_update(x_ref[...], xa_ref[...], xb_ref[...], s_ref[...],
                   bg_ref[...], nu)
    x_out[...] = xn
    p = jnp.concatenate([xn, xa_ref[...], xb_ref[...],
                         jnp.zeros((xn.shape[0], 55), F32)], axis=1)
    ts_ref[...] = jnp.concatenate([msn_ref[...], p], axis=1)
    td_ref[...] = jnp.concatenate([mdn_ref[...], p], axis=1)


def _final_body(x_ref, xa_ref, xb_ref, s_ref, bg_ref, n0_ref, n1_ref, out_ref):
    nu = n0_ref[:, 0:3] + n1_ref[:, 0:3]
    xn = _x_update(x_ref[...], xa_ref[...], xb_ref[...], s_ref[...],
                   bg_ref[...], nu)
    out_ref[...] = xn - jnp.mean(xn, axis=0, keepdims=True)


def _edge_spec(width):
    return pl.BlockSpec((BE, width), lambda i: (i, 0))


def _node_spec(width):
    return pl.BlockSpec((BN, width), lambda i: (i, 0))


def kernel(x_t, xA_pos, xB_pos, s, t, Z, edge_index, is_bond_A, is_bond_B, params):
    src = edge_index[0].astype(jnp.int32)
    dst = edge_index[1].astype(jnp.int32)
    pad = E_PAD - E
    srcp = jnp.concatenate([src, jnp.zeros((pad,), jnp.int32)]).reshape(NW, CPW, LCH)
    dstp = jnp.concatenate([dst, jnp.zeros((pad,), jnp.int32)]).reshape(NW, CPW, LCH)
    dsts = jnp.concatenate([dst, jnp.full((pad,), N, jnp.int32)]).reshape(NW, CPW, LCH)
    ib2 = jnp.concatenate(
        [jnp.stack([is_bond_A, is_bond_B], axis=1), jnp.zeros((pad, 2), F32)], axis=0)
    zeros128 = jnp.zeros((N_ACC, 128), F32)
    s2 = s[:, None]
    t2 = t[:, None]
    zf = Z.astype(F32)[:, None]

    def w2d(b):
        return b.reshape(1, -1)

    P = params
    NG = N // BN
    EG = E_PAD // BE

    wm0, bm0 = P["message"][0][0]
    pro_in = [zf, s2, t2, x_t, xA_pos, xB_pos,
              P["info"][0][0], w2d(P["info"][0][1]), P["info"][1][0], w2d(P["info"][1][1]),
              P["embA"][0][0], w2d(P["embA"][0][1]), P["embA"][1][0], w2d(P["embA"][1][1]),
              P["embB"][0][0], w2d(P["embB"][0][1]), P["embB"][1][0], w2d(P["embB"][1][1]),
              wm0[:STATE], wm0[STATE:2 * STATE]]
    h, tabs, tabd = pl.pallas_call(
        _prologue_body,
        grid=(NG,),
        in_specs=[_node_spec(1), _node_spec(1), _node_spec(1),
                  _node_spec(3), _node_spec(3), _node_spec(3)] +
                 [_full(a) for a in pro_in[6:]],
        out_specs=[_node_spec(STATE), _node_spec(128), _node_spec(128)],
        out_shape=[jax.ShapeDtypeStruct((N, STATE), F32),
                   jax.ShapeDtypeStruct((N, 128), F32),
                   jax.ShapeDtypeStruct((N, 128), F32)],
        name="prologue",
    )(*pro_in)

    x = x_t
    out = None
    for l in range(2):
        wm1, bm1 = P["message"][l][0]
        wm2, bm2 = P["message"][l][1]
        wm3, bm3 = P["message"][l][2]
        ws1, bs1 = P["state"][l][0]
        ws2, bs2 = P["state"][l][1]
        ws3, bs3 = P["state"][l][2]
        wa1, ba1 = P["alpha"][l][0]
        wa2, ba2 = P["alpha"][l][1]
        wa3, ba3 = P["alpha"][l][2]

        gs, gd = _get_gather(128)(tabs, tabd, srcp, dstp)

        msg_w = [wm1[2 * STATE:], w2d(bm1), wm2, w2d(bm2), wm3, w2d(bm3)]
        msg = pl.pallas_call(
            _edge_msg_body,
            grid=(EG,),
            in_specs=[_edge_spec(128), _edge_spec(128), _edge_spec(2)] +
                     [_full(a) for a in msg_w],
            out_specs=_edge_spec(128),
            out_shape=jax.ShapeDtypeStruct((E_PAD, 128), F32),
            name="edge_msg",
        )(gs, gd, ib2, *msg_w)

        nmp = _get_scatter(128)(msg, dsts, zeros128)
        nm0, nm1 = nmp[0], nmp[1]

        node_w = [ws1[:STATE], ws1[STATE:], w2d(bs1), ws2, w2d(bs2), ws3, w2d(bs3),
                  wa1[:STATE], wa1[STATE:2 * STATE]]
        for nm_ in ("beta", "gamma"):
            for li in range(3):
                node_w.append(P[nm_][l][li][0])
                node_w.append(w2d(P[nm_][l][li][1]))
        if l == 0:
            wmn = P["message"][1][0][0]
            node_w += [wmn[:STATE], wmn[STATE:2 * STATE]]
            h, taba, bgv, msn, mdn = pl.pallas_call(
                _node_update_body0,
                grid=(NG,),
                in_specs=[_node_spec(STATE),
                          pl.BlockSpec((BN, 128), lambda i: (i, 0)),
                          pl.BlockSpec((BN, 128), lambda i: (i, 0))] +
                         [_full(a) for a in node_w],
                out_specs=[_node_spec(STATE), _node_spec(128),
                           _node_spec(2), _node_spec(64), _node_spec(64)],
                out_shape=[jax.ShapeDtypeStruct((N, STATE), F32),
                           jax.ShapeDtypeStruct((N, 128), F32),
                           jax.ShapeDtypeStruct((N, 2), F32),
                           jax.ShapeDtypeStruct((N, 64), F32),
                           jax.ShapeDtypeStruct((N, 64), F32)],
                name="node_update0",
            )(h, nm0, nm1, *node_w)
        else:
            h, taba, bgv = pl.pallas_call(
                _node_update_body1,
                grid=(NG,),
                in_specs=[_node_spec(STATE),
                          pl.BlockSpec((BN, 128), lambda i: (i, 0)),
                          pl.BlockSpec((BN, 128), lambda i: (i, 0))] +
                         [_full(a) for a in node_w],
                out_specs=[_node_spec(STATE), _node_spec(128),
                           _node_spec(2)],
                out_shape=[jax.ShapeDtypeStruct((N, STATE), F32),
                           jax.ShapeDtypeStruct((N, 128), F32),
                           jax.ShapeDtypeStruct((N, 2), F32)],
                name="node_update1",
            )(h, nm0, nm1, *node_w)

        ga, gb = _get_gather(128)(taba, taba, srcp, dstp)

        al_w = [wa1[2 * STATE:], w2d(ba1), wa2, w2d(ba2), wa3, w2d(ba3)]
        av = pl.pallas_call(
            _edge_alpha_body,
            grid=(EG,),
            in_specs=[_edge_spec(128), _edge_spec(128), _edge_spec(128),
                      _edge_spec(128), _edge_spec(2)] +
                     [_full(a) for a in al_w],
            out_specs=_edge_spec(128),
            out_shape=jax.ShapeDtypeStruct((E_PAD, 128), F32),
            name="edge_alpha",
        )(ga, gb, gs, gd, ib2, *al_w)

        nup = _get_scatter(128)(av, dsts, zeros128)
        nu0, nu1 = nup[0], nup[1]

        if l == 0:
            tabs, tabd, x = pl.pallas_call(
                _transition_body,
                grid=(NG,),
                in_specs=[_node_spec(3), _node_spec(3), _node_spec(3),
                          _node_spec(1), _node_spec(2),
                          pl.BlockSpec((BN, 128), lambda i: (i, 0)),
                          pl.BlockSpec((BN, 128), lambda i: (i, 0)),
                          _node_spec(64), _node_spec(64)],
                out_specs=[_node_spec(128), _node_spec(128), _node_spec(3)],
                out_shape=[jax.ShapeDtypeStruct((N, 128), F32),
                           jax.ShapeDtypeStruct((N, 128), F32),
                           jax.ShapeDtypeStruct((N, 3), F32)],
                name="transition",
            )(x, xA_pos, xB_pos, s2, bgv, nu0, nu1, msn, mdn)
        else:
            out = pl.pallas_call(
                _final_body,
                grid=(1,),
                in_specs=[pl.BlockSpec((N, 3), lambda i: (0, 0)),
                          pl.BlockSpec((N, 3), lambda i: (0, 0)),
                          pl.BlockSpec((N, 3), lambda i: (0, 0)),
                          pl.BlockSpec((N, 1), lambda i: (0, 0)),
                          pl.BlockSpec((N, 2), lambda i: (0, 0)),
                          pl.BlockSpec((N, 128), lambda i: (0, 0)),
                          pl.BlockSpec((N, 128), lambda i: (0, 0))],
                out_specs=pl.BlockSpec((N, 3), lambda i: (0, 0)),
                out_shape=jax.ShapeDtypeStruct((N, 3), F32),
                name="final",
            )(x, xA_pos, xB_pos, s2, bgv, nu0, nu1)
    return out


# speedup vs baseline: 4.4990x; 1.6143x over previous
import functools

import jax
import jax.numpy as jnp
from jax import lax
from jax.experimental import pallas as pl
from jax.experimental.pallas import tpu as pltpu
from jax.experimental.pallas import tpu_sc as plsc

F32 = jnp.float32
N = 10000
E = 160000
STATE = 224
NFREQ = 8
NRBF = 10
DCUT = 5.0

NW = 32
NC = 2
NS = 16
LCH = 128
E_PAD = 163840
CPW = E_PAD // (NW * LCH)
CPW2 = E_PAD // (NS * LCH)
N_ACC = 10112
BE = 2048
BN = 1000


def _gelu(x):
    return jax.nn.gelu(x)


def _make_sc_gather(width):
    del width
    mesh = plsc.VectorSubcoreMesh(core_axis_name="c", subcore_axis_name="s",
                                  num_cores=NC, num_subcores=NS)
    trows = N_ACC // NS

    @functools.partial(
        pl.kernel,
        out_type=(jax.ShapeDtypeStruct((E_PAD, 128), F32),
                  jax.ShapeDtypeStruct((E_PAD, 128), F32)),
        mesh=mesh,
        scratch_types=[
            pltpu.VMEM((CPW2, LCH), jnp.int32),
            pltpu.VMEM((2, LCH, 128), F32),
            pltpu.VMEM_SHARED((N_ACC, 128), F32),
        ] + [pltpu.SemaphoreType.DMA] * 4,
        name="sc_gather_spmem",
    )
    def gather(tab_s, tab_d, srcw, dstw, out_s, out_d,
               idxv, bufs, stab, g0, g1, o0, o1):
        cid = lax.axis_index("c")
        sid = lax.axis_index("s")
        gsem = (g0, g1)
        osem = (o0, o1)

        @pl.when(cid == 0)
        def _():
            pltpu.sync_copy(tab_s.at[pl.ds(sid * trows, trows)],
                            stab.at[pl.ds(sid * trows, trows)])
            pltpu.sync_copy(srcw.at[sid], idxv)

        @pl.when(cid == 1)
        def _():
            pltpu.sync_copy(tab_d.at[pl.ds(sid * trows, trows)],
                            stab.at[pl.ds(sid * trows, trows)])
            pltpu.sync_copy(dstw.at[sid], idxv)

        plsc.subcore_barrier()
        for b in range(2):
            pltpu.async_copy(stab.at[idxv.at[b]], bufs.at[b], gsem[b])

        def body(p, carry):
            for b in range(2):
                jc = 2 * p + b
                base = (sid * CPW2 + jc) * LCH
                pltpu.make_async_copy(stab.at[idxv.at[jc]],
                                      bufs.at[b], gsem[b]).wait()

                @pl.when(cid == 0)
                def _():
                    pltpu.async_copy(bufs.at[b], out_s.at[pl.ds(base, LCH)],
                                     osem[b])

                @pl.when(cid == 1)
                def _():
                    pltpu.async_copy(bufs.at[b], out_d.at[pl.ds(base, LCH)],
                                     osem[b])

                @pl.when(p < CPW2 // 2 - 1)
                def _():
                    pltpu.make_async_copy(bufs.at[b],
                                          out_s.at[pl.ds(base, LCH)],
                                          osem[b]).wait()
                    pltpu.async_copy(stab.at[idxv.at[jc + 2]],
                                     bufs.at[b], gsem[b])
            return carry

        lax.fori_loop(0, CPW2 // 2, body, 0)
        for b in range(2):
            base = (sid * CPW2 + CPW2 - 2 + b) * LCH
            pltpu.make_async_copy(bufs.at[b], out_s.at[pl.ds(base, LCH)],
                                  osem[b]).wait()

    return gather


def _make_sc_scatter(width):
    del width
    mesh = plsc.VectorSubcoreMesh(core_axis_name="c", subcore_axis_name="s",
                                  num_cores=NC, num_subcores=NS)
    rows = N_ACC // NS

    @functools.partial(
        pl.kernel,
        out_type=jax.ShapeDtypeStruct((2, N_ACC, 128), F32),
        mesh=mesh,
        scratch_types=[
            pltpu.VMEM((LCH,), jnp.int32),
            pltpu.VMEM((LCH, 128), F32),
            pltpu.VMEM_SHARED((N_ACC, 128), F32),
        ],
        name="sc_scatter128",
    )
    def scatter(vals, dstw, zeros_hbm, out, idxc, buf, acc):
        cid = lax.axis_index("c")
        sid = lax.axis_index("s")
        wid = sid * NC + cid
        pltpu.sync_copy(zeros_hbm.at[pl.ds(sid * rows, rows)],
                        acc.at[pl.ds(sid * rows, rows)])
        plsc.subcore_barrier()

        def body(j, carry):
            base = (wid * CPW + j) * LCH
            pltpu.sync_copy(dstw.at[wid, j], idxc)
            pltpu.sync_copy(vals.at[pl.ds(base, LCH)], buf)
            pltpu.sync_copy(buf, acc.at[idxc], add=True)
            return carry

        lax.fori_loop(0, CPW, body, 0)
        plsc.subcore_barrier()
        pltpu.sync_copy(acc.at[pl.ds(sid * rows, rows)],
                        out.at[cid, pl.ds(sid * rows, rows)])

    return scatter


_get_gather = functools.lru_cache(None)(_make_sc_gather)
_get_scatter = functools.lru_cache(None)(_make_sc_scatter)


def _full(a):
    return pl.BlockSpec(a.shape, lambda i: (0,) * a.ndim)


def _rbf_feats(d):
    c = lax.broadcasted_iota(jnp.int32, (1, NRBF), 1).astype(F32) * (DCUT / (NRBF - 1))
    inv2s2 = 1.0 / (2.0 * (DCUT / NRBF) ** 2)
    return jnp.exp(-((d - c) ** 2) * inv2s2)


def _edge_feats(gs, gd, ib2):
    xs = gs[:, 64:73]
    xd = gd[:, 64:73]
    dxyz = xs - xd
    d0 = dxyz[:, 0:3]
    d2 = jnp.sum(d0 * d0, axis=1, keepdims=True)
    dist = jnp.sqrt(d2 + 1e-12)
    dA3 = dxyz[:, 3:6]
    dA = jnp.sqrt(jnp.sum(dA3 * dA3, axis=1, keepdims=True) + 1e-12)
    dB3 = dxyz[:, 6:9]
    dB = jnp.sqrt(jnp.sum(dB3 * dB3, axis=1, keepdims=True) + 1e-12)
    ef = jnp.concatenate([ib2, dist, dist * dist, dA, dB, dA - dB,
                          _rbf_feats(dist), _rbf_feats(dA), _rbf_feats(dB)],
                         axis=1)
    return ef, d0


def _edge_msg_body(gs_ref, gd_ref, ib_ref, w1e, b1, w2, b2, w3, b3, out_ref):
    gs = gs_ref[...]
    gd = gd_ref[...]
    ef, _ = _edge_feats(gs, gd, ib_ref[...])
    g = gs[:, :64] + gd[:, :64]
    u = _gelu(g + jnp.dot(ef, w1e[...]) + b1[...])
    u = _gelu(jnp.dot(u, w2[...]) + b2[...])
    m = jnp.dot(u, w3[...]) + b3[...]
    out_ref[...] = jnp.concatenate([m, jnp.zeros((m.shape[0], 64), F32)], axis=1)


def _edge_alpha_body(ga_ref, gb_ref, gs_ref, gd_ref, ib_ref,
                     w1e, b1, w2, b2, w3, b3, out_ref):
    gs = gs_ref[...]
    gd = gd_ref[...]
    ef, d0 = _edge_feats(gs, gd, ib_ref[...])
    g = ga_ref[:, :64] + gb_ref[:, 64:128]
    u = _gelu(g + jnp.dot(ef, w1e[...]) + b1[...])
    u = _gelu(jnp.dot(u, w2[...]) + b2[...])
    alpha = jnp.dot(u, w3[...]) + b3[...]
    av = alpha * d0
    out_ref[...] = jnp.concatenate(
        [av, jnp.zeros((av.shape[0], 125), F32)], axis=1)


def _prologue_body(zf_ref, s_ref, t_ref, x_ref, xa_ref, xb_ref,
                   wi1, bi1, wi2, bi2, wa1, ba1, wa2, ba2,
                   wb1, bb1, wb2, bb2, wms, wmd,
                   h_ref, ts_ref, td_ref):
    tab = jnp.concatenate([
        jnp.dot(_gelu(wi1[...] + bi1[...]), wi2[...]) + bi2[...],
        jnp.dot(_gelu(wa1[...] + ba1[...]), wa2[...]) + ba2[...],
        jnp.dot(_gelu(wb1[...] + bb1[...]), wb2[...]) + bb2[...],
    ], axis=1)
    z = zf_ref[...]
    oh = (z == lax.broadcasted_iota(jnp.int32, (z.shape[0], 10), 1).astype(F32)).astype(F32)
    emb = jnp.dot(oh, tab)
    k = jnp.exp2(lax.broadcasted_iota(jnp.int32, (1, NFREQ), 1).astype(F32)) * jnp.pi
    angs = s_ref[...] * k
    angt = t_ref[...] * k
    h = jnp.concatenate([emb, jnp.sin(angs), jnp.cos(angs),
                         jnp.sin(angt), jnp.cos(angt)], axis=1)
    h_ref[...] = h
    p = jnp.concatenate([x_ref[...], xa_ref[...], xb_ref[...],
                         jnp.zeros((h.shape[0], 55), F32)], axis=1)
    ts_ref[...] = jnp.concatenate([jnp.dot(h, wms[...]), p], axis=1)
    td_ref[...] = jnp.concatenate([jnp.dot(h, wmd[...]), p], axis=1)


def _node_update_body0(h_ref, n0_ref, n1_ref,
                       ws1a, ws1b, bs1, ws2, bs2, ws3, bs3,
                       wa1s, wa1d,
                       wbe1, bbe1, wbe2, bbe2, wbe3, bbe3,
                       wg1, bg1, wg2, bg2, wg3, bg3,
                       wmsn, wmdn,
                       h_out, ta_out, bg_out, msn_out, mdn_out):
    h = h_ref[...]
    nm = n0_ref[:, :64] + n1_ref[:, :64]
    v = _gelu(jnp.dot(h, ws1a[...]) + jnp.dot(nm, ws1b[...]) + bs1[...])
    v = _gelu(jnp.dot(v, ws2[...]) + bs2[...])
    hn = h + jnp.dot(v, ws3[...]) + bs3[...]
    h_out[...] = hn
    ta_out[...] = jnp.concatenate(
        [jnp.dot(hn, wa1s[...]), jnp.dot(hn, wa1d[...])], axis=1)
    ub = _gelu(jnp.dot(hn, wbe1[...]) + bbe1[...])
    ub = _gelu(jnp.dot(ub, wbe2[...]) + bbe2[...])
    beta = jnp.dot(ub, wbe3[...]) + bbe3[...]
    ug = _gelu(jnp.dot(hn, wg1[...]) + bg1[...])
    ug = _gelu(jnp.dot(ug, wg2[...]) + bg2[...])
    gamma = jnp.dot(ug, wg3[...]) + bg3[...]
    bg_out[...] = jnp.concatenate([beta, gamma], axis=1)
    msn_out[...] = jnp.dot(hn, wmsn[...])
    mdn_out[...] = jnp.dot(hn, wmdn[...])


def _node_update_body1(h_ref, n0_ref, n1_ref,
                       ws1a, ws1b, bs1, ws2, bs2, ws3, bs3,
                       wa1s, wa1d,
                       wbe1, bbe1, wbe2, bbe2, wbe3, bbe3,
                       wg1, bg1, wg2, bg2, wg3, bg3,
                       h_out, ta_out, bg_out):
    h = h_ref[...]
    nm = n0_ref[:, :64] + n1_ref[:, :64]
    v = _gelu(jnp.dot(h, ws1a[...]) + jnp.dot(nm, ws1b[...]) + bs1[...])
    v = _gelu(jnp.dot(v, ws2[...]) + bs2[...])
    hn = h + jnp.dot(v, ws3[...]) + bs3[...]
    h_out[...] = hn
    ta_out[...] = jnp.concatenate(
        [jnp.dot(hn, wa1s[...]), jnp.dot(hn, wa1d[...])], axis=1)
    ub = _gelu(jnp.dot(hn, wbe1[...]) + bbe1[...])
    ub = _gelu(jnp.dot(ub, wbe2[...]) + bbe2[...])
    beta = jnp.dot(ub, wbe3[...]) + bbe3[...]
    ug = _gelu(jnp.dot(hn, wg1[...]) + bg1[...])
    ug = _gelu(jnp.dot(ug, wg2[...]) + bg2[...])
    gamma = jnp.dot(ug, wg3[...]) + bg3[...]
    bg_out[...] = jnp.concatenate([beta, gamma], axis=1)


def _x_update(x, xa, xb, s2, bg, nu):
    beta = bg[:, 0:1]
    gamma = bg[:, 1:2]
    return x + nu + beta * (1.0 - s2) * (xa - x) + gamma * s2 * (xb - x)


def _transition_body(x_ref, xa_ref, xb_ref, s_ref, bg_ref, n0_ref, n1_ref,
                     msn_ref, mdn_ref, ts_ref, td_ref, x_out):
    nu = n0_ref[:, 0:3] + n1_ref[:, 0:3]
    xn = _x_update(x_ref[...], xa_ref[...], xb_ref[...], s_ref[...],
                   bg_ref[...], nu)
    x_out[...] = xn
    p = jnp.concatenate([xn, xa_ref[...], xb_ref[...],
                         jnp.zeros((xn.shape[0], 55), F32)], axis=1)
    ts_ref[...] = jnp.concatenate([msn_ref[...], p], axis=1)
    td_ref[...] = jnp.concatenate([mdn_ref[...], p], axis=1)


def _final_body(x_ref, xa_ref, xb_ref, s_ref, bg_ref, n0_ref, n1_ref, out_ref):
    nu = n0_ref[:, 0:3] + n1_ref[:, 0:3]
    xn = _x_update(x_ref[...], xa_ref[...], xb_ref[...], s_ref[...],
                   bg_ref[...], nu)
    out_ref[...] = xn - jnp.mean(xn, axis=0, keepdims=True)


def _edge_spec(width):
    return pl.BlockSpec((BE, width), lambda i: (i, 0))


def _node_spec(width):
    return pl.BlockSpec((BN, width), lambda i: (i, 0))


def kernel(x_t, xA_pos, xB_pos, s, t, Z, edge_index, is_bond_A, is_bond_B, params):
    src = edge_index[0].astype(jnp.int32)
    dst = edge_index[1].astype(jnp.int32)
    pad = E_PAD - E
    srcp = jnp.concatenate([src, jnp.zeros((pad,), jnp.int32)]).reshape(NS, CPW2, LCH)
    dstp = jnp.concatenate([dst, jnp.zeros((pad,), jnp.int32)]).reshape(NS, CPW2, LCH)
    dsts = jnp.concatenate([dst, jnp.full((pad,), N, jnp.int32)]).reshape(NW, CPW, LCH)
    ib2 = jnp.concatenate(
        [jnp.stack([is_bond_A, is_bond_B], axis=1), jnp.zeros((pad, 2), F32)], axis=0)
    zeros128 = jnp.zeros((N_ACC, 128), F32)
    s2 = s[:, None]
    t2 = t[:, None]
    zf = Z.astype(F32)[:, None]

    def w2d(b):
        return b.reshape(1, -1)

    P = params
    NG = N // BN
    EG = E_PAD // BE

    wm0, bm0 = P["message"][0][0]
    pro_in = [zf, s2, t2, x_t, xA_pos, xB_pos,
              P["info"][0][0], w2d(P["info"][0][1]), P["info"][1][0], w2d(P["info"][1][1]),
              P["embA"][0][0], w2d(P["embA"][0][1]), P["embA"][1][0], w2d(P["embA"][1][1]),
              P["embB"][0][0], w2d(P["embB"][0][1]), P["embB"][1][0], w2d(P["embB"][1][1]),
              wm0[:STATE], wm0[STATE:2 * STATE]]
    h, tabs, tabd = pl.pallas_call(
        _prologue_body,
        grid=(NG,),
        in_specs=[_node_spec(1), _node_spec(1), _node_spec(1),
                  _node_spec(3), _node_spec(3), _node_spec(3)] +
                 [_full(a) for a in pro_in[6:]],
        out_specs=[_node_spec(STATE), _node_spec(128), _node_spec(128)],
        out_shape=[jax.ShapeDtypeStruct((N, STATE), F32),
                   jax.ShapeDtypeStruct((N_ACC, 128), F32),
                   jax.ShapeDtypeStruct((N_ACC, 128), F32)],
        name="prologue",
    )(*pro_in)

    x = x_t
    out = None
    for l in range(2):
        wm1, bm1 = P["message"][l][0]
        wm2, bm2 = P["message"][l][1]
        wm3, bm3 = P["message"][l][2]
        ws1, bs1 = P["state"][l][0]
        ws2, bs2 = P["state"][l][1]
        ws3, bs3 = P["state"][l][2]
        wa1, ba1 = P["alpha"][l][0]
        wa2, ba2 = P["alpha"][l][1]
        wa3, ba3 = P["alpha"][l][2]

        gs, gd = _get_gather(128)(tabs, tabd, srcp, dstp)

        msg_w = [wm1[2 * STATE:], w2d(bm1), wm2, w2d(bm2), wm3, w2d(bm3)]
        msg = pl.pallas_call(
            _edge_msg_body,
            grid=(EG,),
            in_specs=[_edge_spec(128), _edge_spec(128), _edge_spec(2)] +
                     [_full(a) for a in msg_w],
            out_specs=_edge_spec(128),
            out_shape=jax.ShapeDtypeStruct((E_PAD, 128), F32),
            name="edge_msg",
        )(gs, gd, ib2, *msg_w)

        nmp = _get_scatter(128)(msg, dsts, zeros128)
        nm0, nm1 = nmp[0], nmp[1]

        node_w = [ws1[:STATE], ws1[STATE:], w2d(bs1), ws2, w2d(bs2), ws3, w2d(bs3),
                  wa1[:STATE], wa1[STATE:2 * STATE]]
        for nm_ in ("beta", "gamma"):
            for li in range(3):
                node_w.append(P[nm_][l][li][0])
                node_w.append(w2d(P[nm_][l][li][1]))
        if l == 0:
            wmn = P["message"][1][0][0]
            node_w += [wmn[:STATE], wmn[STATE:2 * STATE]]
            h, taba, bgv, msn, mdn = pl.pallas_call(
                _node_update_body0,
                grid=(NG,),
                in_specs=[_node_spec(STATE),
                          pl.BlockSpec((BN, 128), lambda i: (i, 0)),
                          pl.BlockSpec((BN, 128), lambda i: (i, 0))] +
                         [_full(a) for a in node_w],
                out_specs=[_node_spec(STATE), _node_spec(128),
                           _node_spec(2), _node_spec(64), _node_spec(64)],
                out_shape=[jax.ShapeDtypeStruct((N, STATE), F32),
                           jax.ShapeDtypeStruct((N_ACC, 128), F32),
                           jax.ShapeDtypeStruct((N, 2), F32),
                           jax.ShapeDtypeStruct((N, 64), F32),
                           jax.ShapeDtypeStruct((N, 64), F32)],
                name="node_update0",
            )(h, nm0, nm1, *node_w)
        else:
            h, taba, bgv = pl.pallas_call(
                _node_update_body1,
                grid=(NG,),
                in_specs=[_node_spec(STATE),
                          pl.BlockSpec((BN, 128), lambda i: (i, 0)),
                          pl.BlockSpec((BN, 128), lambda i: (i, 0))] +
                         [_full(a) for a in node_w],
                out_specs=[_node_spec(STATE), _node_spec(128),
                           _node_spec(2)],
                out_shape=[jax.ShapeDtypeStruct((N, STATE), F32),
                           jax.ShapeDtypeStruct((N_ACC, 128), F32),
                           jax.ShapeDtypeStruct((N, 2), F32)],
                name="node_update1",
            )(h, nm0, nm1, *node_w)

        ga, gb = _get_gather(128)(taba, taba, srcp, dstp)

        al_w = [wa1[2 * STATE:], w2d(ba1), wa2, w2d(ba2), wa3, w2d(ba3)]
        av = pl.pallas_call(
            _edge_alpha_body,
            grid=(EG,),
            in_specs=[_edge_spec(128), _edge_spec(128), _edge_spec(128),
                      _edge_spec(128), _edge_spec(2)] +
                     [_full(a) for a in al_w],
            out_specs=_edge_spec(128),
            out_shape=jax.ShapeDtypeStruct((E_PAD, 128), F32),
            name="edge_alpha",
        )(ga, gb, gs, gd, ib2, *al_w)

        nup = _get_scatter(128)(av, dsts, zeros128)
        nu0, nu1 = nup[0], nup[1]

        if l == 0:
            tabs, tabd, x = pl.pallas_call(
                _transition_body,
                grid=(NG,),
                in_specs=[_node_spec(3), _node_spec(3), _node_spec(3),
                          _node_spec(1), _node_spec(2),
                          pl.BlockSpec((BN, 128), lambda i: (i, 0)),
                          pl.BlockSpec((BN, 128), lambda i: (i, 0)),
                          _node_spec(64), _node_spec(64)],
                out_specs=[_node_spec(128), _node_spec(128), _node_spec(3)],
                out_shape=[jax.ShapeDtypeStruct((N_ACC, 128), F32),
                           jax.ShapeDtypeStruct((N_ACC, 128), F32),
                           jax.ShapeDtypeStruct((N, 3), F32)],
                name="transition",
            )(x, xA_pos, xB_pos, s2, bgv, nu0, nu1, msn, mdn)
        else:
            out = pl.pallas_call(
                _final_body,
                grid=(1,),
                in_specs=[pl.BlockSpec((N, 3), lambda i: (0, 0)),
                          pl.BlockSpec((N, 3), lambda i: (0, 0)),
                          pl.BlockSpec((N, 3), lambda i: (0, 0)),
                          pl.BlockSpec((N, 1), lambda i: (0, 0)),
                          pl.BlockSpec((N, 2), lambda i: (0, 0)),
                          pl.BlockSpec((N, 128), lambda i: (0, 0)),
                          pl.BlockSpec((N, 128), lambda i: (0, 0))],
                out_specs=pl.BlockSpec((N, 3), lambda i: (0, 0)),
                out_shape=jax.ShapeDtypeStruct((N, 3), F32),
                name="final",
            )(x, xA_pos, xB_pos, s2, bgv, nu0, nu1)
    return out


# speedup vs baseline: 5.3287x; 1.1844x over previous
import functools

import jax
import jax.numpy as jnp
from jax import lax
from jax.experimental import pallas as pl
from jax.experimental.pallas import tpu as pltpu
from jax.experimental.pallas import tpu_sc as plsc

F32 = jnp.float32
N = 10000
E = 160000
STATE = 224
NFREQ = 8
NRBF = 10
DCUT = 5.0

NW = 32
NC = 2
NS = 16
LCH = 128
E_PAD = 163840
CPW = E_PAD // (NW * LCH)
CPW2 = E_PAD // (NS * LCH)
N_ACC = 10112
BE = 2048
BN = 1000


def _gelu(x):
    return jax.nn.gelu(x)


def _make_sc_gather(width):
    del width
    mesh = plsc.VectorSubcoreMesh(core_axis_name="c", subcore_axis_name="s",
                                  num_cores=NC, num_subcores=NS)
    trows = N_ACC // NS

    @functools.partial(
        pl.kernel,
        out_type=(jax.ShapeDtypeStruct((E_PAD, 128), F32),
                  jax.ShapeDtypeStruct((E_PAD, 128), F32)),
        mesh=mesh,
        scratch_types=[
            pltpu.VMEM((CPW2, LCH), jnp.int32),
            pltpu.VMEM((2, LCH, 128), F32),
            pltpu.VMEM_SHARED((N_ACC, 128), F32),
        ] + [pltpu.SemaphoreType.DMA] * 4,
        name="sc_gather_spmem",
    )
    def gather(tab_s, tab_d, srcw, dstw, out_s, out_d,
               idxv, bufs, stab, g0, g1, o0, o1):
        cid = lax.axis_index("c")
        sid = lax.axis_index("s")
        gsem = (g0, g1)
        osem = (o0, o1)

        @pl.when(cid == 0)
        def _():
            pltpu.sync_copy(tab_s.at[pl.ds(sid * trows, trows)],
                            stab.at[pl.ds(sid * trows, trows)])
            pltpu.sync_copy(srcw.at[sid], idxv)

        @pl.when(cid == 1)
        def _():
            pltpu.sync_copy(tab_d.at[pl.ds(sid * trows, trows)],
                            stab.at[pl.ds(sid * trows, trows)])
            pltpu.sync_copy(dstw.at[sid], idxv)

        plsc.subcore_barrier()
        for b in range(2):
            pltpu.async_copy(stab.at[idxv.at[b]], bufs.at[b], gsem[b])

        def body(p, carry):
            for b in range(2):
                jc = 2 * p + b
                base = (sid * CPW2 + jc) * LCH
                pltpu.make_async_copy(stab.at[idxv.at[jc]],
                                      bufs.at[b], gsem[b]).wait()

                @pl.when(cid == 0)
                def _():
                    pltpu.async_copy(bufs.at[b], out_s.at[pl.ds(base, LCH)],
                                     osem[b])

                @pl.when(cid == 1)
                def _():
                    pltpu.async_copy(bufs.at[b], out_d.at[pl.ds(base, LCH)],
                                     osem[b])

                @pl.when(p < CPW2 // 2 - 1)
                def _():
                    pltpu.make_async_copy(bufs.at[b],
                                          out_s.at[pl.ds(base, LCH)],
                                          osem[b]).wait()
                    pltpu.async_copy(stab.at[idxv.at[jc + 2]],
                                     bufs.at[b], gsem[b])
            return carry

        lax.fori_loop(0, CPW2 // 2, body, 0)
        for b in range(2):
            base = (sid * CPW2 + CPW2 - 2 + b) * LCH
            pltpu.make_async_copy(bufs.at[b], out_s.at[pl.ds(base, LCH)],
                                  osem[b]).wait()

    return gather


def _make_sc_scatter(width):
    del width
    mesh = plsc.VectorSubcoreMesh(core_axis_name="c", subcore_axis_name="s",
                                  num_cores=NC, num_subcores=NS)
    rows = N_ACC // NS

    @functools.partial(
        pl.kernel,
        out_type=jax.ShapeDtypeStruct((2, N_ACC, 128), F32),
        mesh=mesh,
        scratch_types=[
            pltpu.VMEM((LCH,), jnp.int32),
            pltpu.VMEM((LCH, 128), F32),
            pltpu.VMEM_SHARED((N_ACC, 128), F32),
        ],
        name="sc_scatter128",
    )
    def scatter(vals, dstw, zeros_hbm, out, idxc, buf, acc):
        cid = lax.axis_index("c")
        sid = lax.axis_index("s")
        wid = sid * NC + cid
        pltpu.sync_copy(zeros_hbm.at[pl.ds(sid * rows, rows)],
                        acc.at[pl.ds(sid * rows, rows)])
        plsc.subcore_barrier()

        def body(j, carry):
            base = (wid * CPW + j) * LCH
            pltpu.sync_copy(dstw.at[wid, j], idxc)
            pltpu.sync_copy(vals.at[pl.ds(base, LCH)], buf)
            pltpu.sync_copy(buf, acc.at[idxc], add=True)
            return carry

        lax.fori_loop(0, CPW, body, 0)
        plsc.subcore_barrier()
        pltpu.sync_copy(acc.at[pl.ds(sid * rows, rows)],
                        out.at[cid, pl.ds(sid * rows, rows)])

    return scatter


_get_gather = functools.lru_cache(None)(_make_sc_gather)
_get_scatter = functools.lru_cache(None)(_make_sc_scatter)


def _full(a):
    return pl.BlockSpec(a.shape, lambda i: (0,) * a.ndim)


def _rbf_feats(d):
    c = lax.broadcasted_iota(jnp.int32, (1, NRBF), 1).astype(F32) * (DCUT / (NRBF - 1))
    inv2s2 = 1.0 / (2.0 * (DCUT / NRBF) ** 2)
    return jnp.exp(-((d - c) ** 2) * inv2s2)


def _edge_feats(gs, gd, ib2):
    xs = gs[:, 64:73]
    xd = gd[:, 64:73]
    dxyz = xs - xd
    d0 = dxyz[:, 0:3]
    d2 = jnp.sum(d0 * d0, axis=1, keepdims=True)
    dist = jnp.sqrt(d2 + 1e-12)
    dA3 = dxyz[:, 3:6]
    dA = jnp.sqrt(jnp.sum(dA3 * dA3, axis=1, keepdims=True) + 1e-12)
    dB3 = dxyz[:, 6:9]
    dB = jnp.sqrt(jnp.sum(dB3 * dB3, axis=1, keepdims=True) + 1e-12)
    ef = jnp.concatenate([ib2, dist, dist * dist, dA, dB, dA - dB,
                          _rbf_feats(dist), _rbf_feats(dA), _rbf_feats(dB)],
                         axis=1)
    return ef, d0


def _edge_msg_body(gs_ref, gd_ref, ib_ref, w1e, b1, w2, b2, w3, b3,
                   out_ref, ef_ref):
    gs = gs_ref[...]
    gd = gd_ref[...]
    ef, d0 = _edge_feats(gs, gd, ib_ref[...])
    g = gs[:, :64] + gd[:, :64]
    u = _gelu(g + jnp.dot(ef, w1e[...]) + b1[...])
    u = _gelu(jnp.dot(u, w2[...]) + b2[...])
    m = jnp.dot(u, w3[...]) + b3[...]
    out_ref[...] = jnp.concatenate([m, jnp.zeros((m.shape[0], 64), F32)], axis=1)
    ef_ref[...] = jnp.concatenate([ef, d0], axis=1)


def _edge_alpha_body(ga_ref, gb_ref, ef_ref, w1e, b1, w2, b2, w3, b3, out_ref):
    ef4 = ef_ref[...]
    ef = ef4[:, :37]
    d0 = ef4[:, 37:40]
    g = ga_ref[:, :64] + gb_ref[:, 64:128]
    u = _gelu(g + jnp.dot(ef, w1e[...]) + b1[...])
    u = _gelu(jnp.dot(u, w2[...]) + b2[...])
    alpha = jnp.dot(u, w3[...]) + b3[...]
    av = alpha * d0
    out_ref[...] = jnp.concatenate(
        [av, jnp.zeros((av.shape[0], 125), F32)], axis=1)


def _prologue_body(zf_ref, s_ref, t_ref, x_ref, xa_ref, xb_ref,
                   wi1, bi1, wi2, bi2, wa1, ba1, wa2, ba2,
                   wb1, bb1, wb2, bb2, wms, wmd,
                   h_ref, ts_ref, td_ref):
    tab = jnp.concatenate([
        jnp.dot(_gelu(wi1[...] + bi1[...]), wi2[...]) + bi2[...],
        jnp.dot(_gelu(wa1[...] + ba1[...]), wa2[...]) + ba2[...],
        jnp.dot(_gelu(wb1[...] + bb1[...]), wb2[...]) + bb2[...],
    ], axis=1)
    z = zf_ref[...]
    oh = (z == lax.broadcasted_iota(jnp.int32, (z.shape[0], 10), 1).astype(F32)).astype(F32)
    emb = jnp.dot(oh, tab)
    k = jnp.exp2(lax.broadcasted_iota(jnp.int32, (1, NFREQ), 1).astype(F32)) * jnp.pi
    angs = s_ref[...] * k
    angt = t_ref[...] * k
    h = jnp.concatenate([emb, jnp.sin(angs), jnp.cos(angs),
                         jnp.sin(angt), jnp.cos(angt)], axis=1)
    h_ref[...] = h
    p = jnp.concatenate([x_ref[...], xa_ref[...], xb_ref[...],
                         jnp.zeros((h.shape[0], 55), F32)], axis=1)
    ts_ref[...] = jnp.concatenate([jnp.dot(h, wms[...]), p], axis=1)
    td_ref[...] = jnp.concatenate([jnp.dot(h, wmd[...]), p], axis=1)


def _node_update_body0(h_ref, n0_ref, n1_ref,
                       ws1a, ws1b, bs1, ws2, bs2, ws3, bs3,
                       wa1s, wa1d,
                       wbe1, bbe1, wbe2, bbe2, wbe3, bbe3,
                       wg1, bg1, wg2, bg2, wg3, bg3,
                       wmsn, wmdn,
                       h_out, ta_out, bg_out, msn_out, mdn_out):
    h = h_ref[...]
    nm = n0_ref[:, :64] + n1_ref[:, :64]
    v = _gelu(jnp.dot(h, ws1a[...]) + jnp.dot(nm, ws1b[...]) + bs1[...])
    v = _gelu(jnp.dot(v, ws2[...]) + bs2[...])
    hn = h + jnp.dot(v, ws3[...]) + bs3[...]
    h_out[...] = hn
    ta_out[...] = jnp.concatenate(
        [jnp.dot(hn, wa1s[...]), jnp.dot(hn, wa1d[...])], axis=1)
    ub = _gelu(jnp.dot(hn, wbe1[...]) + bbe1[...])
    ub = _gelu(jnp.dot(ub, wbe2[...]) + bbe2[...])
    beta = jnp.dot(ub, wbe3[...]) + bbe3[...]
    ug = _gelu(jnp.dot(hn, wg1[...]) + bg1[...])
    ug = _gelu(jnp.dot(ug, wg2[...]) + bg2[...])
    gamma = jnp.dot(ug, wg3[...]) + bg3[...]
    bg_out[...] = jnp.concatenate([beta, gamma], axis=1)
    msn_out[...] = jnp.dot(hn, wmsn[...])
    mdn_out[...] = jnp.dot(hn, wmdn[...])


def _node_update_body1(h_ref, n0_ref, n1_ref,
                       ws1a, ws1b, bs1, ws2, bs2, ws3, bs3,
                       wa1s, wa1d,
                       wbe1, bbe1, wbe2, bbe2, wbe3, bbe3,
                       wg1, bg1, wg2, bg2, wg3, bg3,
                       h_out, ta_out, bg_out):
    h = h_ref[...]
    nm = n0_ref[:, :64] + n1_ref[:, :64]
    v = _gelu(jnp.dot(h, ws1a[...]) + jnp.dot(nm, ws1b[...]) + bs1[...])
    v = _gelu(jnp.dot(v, ws2[...]) + bs2[...])
    hn = h + jnp.dot(v, ws3[...]) + bs3[...]
    h_out[...] = hn
    ta_out[...] = jnp.concatenate(
        [jnp.dot(hn, wa1s[...]), jnp.dot(hn, wa1d[...])], axis=1)
    ub = _gelu(jnp.dot(hn, wbe1[...]) + bbe1[...])
    ub = _gelu(jnp.dot(ub, wbe2[...]) + bbe2[...])
    beta = jnp.dot(ub, wbe3[...]) + bbe3[...]
    ug = _gelu(jnp.dot(hn, wg1[...]) + bg1[...])
    ug = _gelu(jnp.dot(ug, wg2[...]) + bg2[...])
    gamma = jnp.dot(ug, wg3[...]) + bg3[...]
    bg_out[...] = jnp.concatenate([beta, gamma], axis=1)


def _x_update(x, xa, xb, s2, bg, nu):
    beta = bg[:, 0:1]
    gamma = bg[:, 1:2]
    return x + nu + beta * (1.0 - s2) * (xa - x) + gamma * s2 * (xb - x)


def _transition_body(x_ref, xa_ref, xb_ref, s_ref, bg_ref, n0_ref, n1_ref,
                     msn_ref, mdn_ref, ts_ref, td_ref, x_out):
    nu = n0_ref[:, 0:3] + n1_ref[:, 0:3]
    xn = _x_update(x_ref[...], xa_ref[...], xb_ref[...], s_ref[...],
                   bg_ref[...], nu)
    x_out[...] = xn
    p = jnp.concatenate([xn, xa_ref[...], xb_ref[...],
                         jnp.zeros((xn.shape[0], 55), F32)], axis=1)
    ts_ref[...] = jnp.concatenate([msn_ref[...], p], axis=1)
    td_ref[...] = jnp.concatenate([mdn_ref[...], p], axis=1)


def _final_body(x_ref, xa_ref, xb_ref, s_ref, bg_ref, n0_ref, n1_ref, out_ref):
    nu = n0_ref[:, 0:3] + n1_ref[:, 0:3]
    xn = _x_update(x_ref[...], xa_ref[...], xb_ref[...], s_ref[...],
                   bg_ref[...], nu)
    out_ref[...] = xn - jnp.mean(xn, axis=0, keepdims=True)


def _edge_spec(width):
    return pl.BlockSpec((BE, width), lambda i: (i, 0))


def _node_spec(width):
    return pl.BlockSpec((BN, width), lambda i: (i, 0))


def kernel(x_t, xA_pos, xB_pos, s, t, Z, edge_index, is_bond_A, is_bond_B, params):
    src = edge_index[0].astype(jnp.int32)
    dst = edge_index[1].astype(jnp.int32)
    pad = E_PAD - E
    srcp = jnp.concatenate([src, jnp.zeros((pad,), jnp.int32)]).reshape(NS, CPW2, LCH)
    dstp = jnp.concatenate([dst, jnp.zeros((pad,), jnp.int32)]).reshape(NS, CPW2, LCH)
    dsts = jnp.concatenate([dst, jnp.full((pad,), N, jnp.int32)]).reshape(NW, CPW, LCH)
    ib2 = jnp.concatenate(
        [jnp.stack([is_bond_A, is_bond_B], axis=1), jnp.zeros((pad, 2), F32)], axis=0)
    zeros128 = jnp.zeros((N_ACC, 128), F32)
    s2 = s[:, None]
    t2 = t[:, None]
    zf = Z.astype(F32)[:, None]

    def w2d(b):
        return b.reshape(1, -1)

    P = params
    NG = N // BN
    EG = E_PAD // BE

    wm0, bm0 = P["message"][0][0]
    pro_in = [zf, s2, t2, x_t, xA_pos, xB_pos,
              P["info"][0][0], w2d(P["info"][0][1]), P["info"][1][0], w2d(P["info"][1][1]),
              P["embA"][0][0], w2d(P["embA"][0][1]), P["embA"][1][0], w2d(P["embA"][1][1]),
              P["embB"][0][0], w2d(P["embB"][0][1]), P["embB"][1][0], w2d(P["embB"][1][1]),
              wm0[:STATE], wm0[STATE:2 * STATE]]
    h, tabs, tabd = pl.pallas_call(
        _prologue_body,
        grid=(NG,),
        in_specs=[_node_spec(1), _node_spec(1), _node_spec(1),
                  _node_spec(3), _node_spec(3), _node_spec(3)] +
                 [_full(a) for a in pro_in[6:]],
        out_specs=[_node_spec(STATE), _node_spec(128), _node_spec(128)],
        out_shape=[jax.ShapeDtypeStruct((N, STATE), F32),
                   jax.ShapeDtypeStruct((N_ACC, 128), F32),
                   jax.ShapeDtypeStruct((N_ACC, 128), F32)],
        name="prologue",
    )(*pro_in)

    x = x_t
    out = None
    for l in range(2):
        wm1, bm1 = P["message"][l][0]
        wm2, bm2 = P["message"][l][1]
        wm3, bm3 = P["message"][l][2]
        ws1, bs1 = P["state"][l][0]
        ws2, bs2 = P["state"][l][1]
        ws3, bs3 = P["state"][l][2]
        wa1, ba1 = P["alpha"][l][0]
        wa2, ba2 = P["alpha"][l][1]
        wa3, ba3 = P["alpha"][l][2]

        gs, gd = _get_gather(128)(tabs, tabd, srcp, dstp)

        msg_w = [wm1[2 * STATE:], w2d(bm1), wm2, w2d(bm2), wm3, w2d(bm3)]
        msg, ef4 = pl.pallas_call(
            _edge_msg_body,
            grid=(EG,),
            in_specs=[_edge_spec(128), _edge_spec(128), _edge_spec(2)] +
                     [_full(a) for a in msg_w],
            out_specs=[_edge_spec(128), _edge_spec(40)],
            out_shape=[jax.ShapeDtypeStruct((E_PAD, 128), F32),
                       jax.ShapeDtypeStruct((E_PAD, 40), F32)],
            name="edge_msg",
        )(gs, gd, ib2, *msg_w)

        nmp = _get_scatter(128)(msg, dsts, zeros128)
        nm0, nm1 = nmp[0], nmp[1]

        node_w = [ws1[:STATE], ws1[STATE:], w2d(bs1), ws2, w2d(bs2), ws3, w2d(bs3),
                  wa1[:STATE], wa1[STATE:2 * STATE]]
        for nm_ in ("beta", "gamma"):
            for li in range(3):
                node_w.append(P[nm_][l][li][0])
                node_w.append(w2d(P[nm_][l][li][1]))
        if l == 0:
            wmn = P["message"][1][0][0]
            node_w += [wmn[:STATE], wmn[STATE:2 * STATE]]
            h, taba, bgv, msn, mdn = pl.pallas_call(
                _node_update_body0,
                grid=(NG,),
                in_specs=[_node_spec(STATE),
                          pl.BlockSpec((BN, 128), lambda i: (i, 0)),
                          pl.BlockSpec((BN, 128), lambda i: (i, 0))] +
                         [_full(a) for a in node_w],
                out_specs=[_node_spec(STATE), _node_spec(128),
                           _node_spec(2), _node_spec(64), _node_spec(64)],
                out_shape=[jax.ShapeDtypeStruct((N, STATE), F32),
                           jax.ShapeDtypeStruct((N_ACC, 128), F32),
                           jax.ShapeDtypeStruct((N, 2), F32),
                           jax.ShapeDtypeStruct((N, 64), F32),
                           jax.ShapeDtypeStruct((N, 64), F32)],
                name="node_update0",
            )(h, nm0, nm1, *node_w)
        else:
            h, taba, bgv = pl.pallas_call(
                _node_update_body1,
                grid=(NG,),
                in_specs=[_node_spec(STATE),
                          pl.BlockSpec((BN, 128), lambda i: (i, 0)),
                          pl.BlockSpec((BN, 128), lambda i: (i, 0))] +
                         [_full(a) for a in node_w],
                out_specs=[_node_spec(STATE), _node_spec(128),
                           _node_spec(2)],
                out_shape=[jax.ShapeDtypeStruct((N, STATE), F32),
                           jax.ShapeDtypeStruct((N_ACC, 128), F32),
                           jax.ShapeDtypeStruct((N, 2), F32)],
                name="node_update1",
            )(h, nm0, nm1, *node_w)

        ga, gb = _get_gather(128)(taba, taba, srcp, dstp)

        al_w = [wa1[2 * STATE:], w2d(ba1), wa2, w2d(ba2), wa3, w2d(ba3)]
        av = pl.pallas_call(
            _edge_alpha_body,
            grid=(EG,),
            in_specs=[_edge_spec(128), _edge_spec(128), _edge_spec(40)] +
                     [_full(a) for a in al_w],
            out_specs=_edge_spec(128),
            out_shape=jax.ShapeDtypeStruct((E_PAD, 128), F32),
            name="edge_alpha",
        )(ga, gb, ef4, *al_w)

        nup = _get_scatter(128)(av, dsts, zeros128)
        nu0, nu1 = nup[0], nup[1]

        if l == 0:
            tabs, tabd, x = pl.pallas_call(
                _transition_body,
                grid=(NG,),
                in_specs=[_node_spec(3), _node_spec(3), _node_spec(3),
                          _node_spec(1), _node_spec(2),
                          pl.BlockSpec((BN, 128), lambda i: (i, 0)),
                          pl.BlockSpec((BN, 128), lambda i: (i, 0)),
                          _node_spec(64), _node_spec(64)],
                out_specs=[_node_spec(128), _node_spec(128), _node_spec(3)],
                out_shape=[jax.ShapeDtypeStruct((N_ACC, 128), F32),
                           jax.ShapeDtypeStruct((N_ACC, 128), F32),
                           jax.ShapeDtypeStruct((N, 3), F32)],
                name="transition",
            )(x, xA_pos, xB_pos, s2, bgv, nu0, nu1, msn, mdn)
        else:
            out = pl.pallas_call(
                _final_body,
                grid=(1,),
                in_specs=[pl.BlockSpec((N, 3), lambda i: (0, 0)),
                          pl.BlockSpec((N, 3), lambda i: (0, 0)),
                          pl.BlockSpec((N, 3), lambda i: (0, 0)),
                          pl.BlockSpec((N, 1), lambda i: (0, 0)),
                          pl.BlockSpec((N, 2), lambda i: (0, 0)),
                          pl.BlockSpec((N, 128), lambda i: (0, 0)),
                          pl.BlockSpec((N, 128), lambda i: (0, 0))],
                out_specs=pl.BlockSpec((N, 3), lambda i: (0, 0)),
                out_shape=jax.ShapeDtypeStruct((N, 3), F32),
                name="final",
            )(x, xA_pos, xB_pos, s2, bgv, nu0, nu1)
    return out
